# baseline (speedup 1.0000x reference)
import functools

import numpy as np
import jax
import jax.numpy as jnp
from jax import lax
from jax.experimental import pallas as pl
from jax.experimental.pallas import tpu as pltpu
from jax.experimental.pallas import tpu_sc as plsc

NC = 2
NS = 16
NW = NC * NS
LANES = 16
CH = 128
RCH = 128
TB = 128
COLW = 128
HID = 256
ALPHA = 0.5

_MESH = dict(core_axis_name="c", subcore_axis_name="s",
             num_cores=NC, num_subcores=NS)


def _silu(x):
    return x * jax.nn.sigmoid(x)


def _chunks(total, step):
    out = []
    off = 0
    while off < total:
        sz = min(step, total - off)
        out.append((off, sz))
        off += sz
    return out


def _deg_ones(srcp, dstp, ones128, n_pad, e_pad):
    half = n_pad // 2
    acc_rows = half + TB
    zstripe = acc_rows // NS
    tr = e_pad // NS
    nch = tr // RCH
    mesh = plsc.VectorSubcoreMesh(**_MESH)

    def body(s_hbm, d_hbm, ones_hbm, od_hbm, id_hbm,
             acco, acci, buf, sv, dv, idxs, idxd):
        cid = lax.axis_index("c")
        sid = lax.axis_index("s")
        _seg_zero(sid, buf, acco, zstripe)
        r0 = sid * zstripe
        for off, sz in _chunks(zstripe, RCH):
            pltpu.sync_copy(buf.at[pl.ds(0, sz)], acci.at[pl.ds(r0 + off, sz)])
        pltpu.sync_copy(ones_hbm, buf)
        ebase = sid * tr
        plsc.subcore_barrier()
        nbase = cid * half

        def step(j, c):
            off = j * RCH
            pltpu.sync_copy(s_hbm.at[pl.ds(ebase + off, RCH)], sv)
            pltpu.sync_copy(d_hbm.at[pl.ds(ebase + off, RCH)], dv)
            _seg_idx(sv, idxs, nbase, half)
            _seg_idx(dv, idxd, nbase, half)
            pltpu.sync_copy(buf, acco.at[idxs.at[0]], add=True)
            pltpu.sync_copy(buf, acci.at[idxd.at[0]], add=True)
            return c

        lax.fori_loop(0, nch, step, 0)
        plsc.subcore_barrier()
        _seg_writeout(cid, sid, 0, buf, acco, od_hbm, half)
        _seg_writeout(cid, sid, 0, buf, acci, id_hbm, half)

    k = pl.kernel(
        body,
        mesh=mesh,
        out_type=(
            jax.ShapeDtypeStruct((n_pad, COLW), jnp.float32),
            jax.ShapeDtypeStruct((n_pad, COLW), jnp.float32),
        ),
        scratch_types=[
            pltpu.VMEM_SHARED((acc_rows, COLW), jnp.float32),
            pltpu.VMEM_SHARED((acc_rows, COLW), jnp.float32),
            pltpu.VMEM((RCH, COLW), jnp.float32),
            pltpu.VMEM((RCH,), jnp.int32),
            pltpu.VMEM((RCH,), jnp.int32),
            pltpu.VMEM((1, RCH), jnp.int32),
            pltpu.VMEM((1, RCH), jnp.int32),
        ],
    )
    return k(srcp, dstp, ones128)


def _gather2(h2, srcg, dstg, e_pad):
    d = h2.shape[1]
    tr = e_pad // NW
    nch = tr // CH
    mesh = plsc.VectorSubcoreMesh(**_MESH)

    def body(h_hbm, s_hbm, d_hbm, hs_hbm, hd_hbm, siv, div, bufa, bufb,
             sema, semb):
        cid = lax.axis_index("c")
        sid = lax.axis_index("s")
        wid = sid * NC + cid
        base = wid * tr
        pltpu.sync_copy(s_hbm.at[pl.ds(base, tr)], siv)
        pltpu.sync_copy(d_hbm.at[pl.ds(base, tr)], div)

        def step(j, c):
            off = j * CH
            cpa = pltpu.async_copy(h_hbm.at[siv.at[pl.ds(off, CH)]], bufa,
                                   sema)
            cpb = pltpu.async_copy(h_hbm.at[div.at[pl.ds(off, CH)]], bufb,
                                   semb)
            cpa.wait()
            pltpu.sync_copy(bufa, hs_hbm.at[pl.ds(base + off, CH)])
            cpb.wait()
            pltpu.sync_copy(bufb, hd_hbm.at[pl.ds(base + off, CH)])
            return c

        lax.fori_loop(0, nch, step, 0)

    k = pl.kernel(
        body,
        mesh=mesh,
        out_type=(
            jax.ShapeDtypeStruct((e_pad, d), jnp.float32),
            jax.ShapeDtypeStruct((e_pad, d), jnp.float32),
        ),
        scratch_types=[
            pltpu.VMEM((tr,), jnp.int32),
            pltpu.VMEM((tr,), jnp.int32),
            pltpu.VMEM((CH, d), jnp.float32),
            pltpu.VMEM((CH, d), jnp.float32),
            pltpu.SemaphoreType.DMA,
            pltpu.SemaphoreType.DMA,
        ],
    )
    return k(h2, srcg, dstg)


def _seg_zero(sid, rows, acc, zstripe):
    zero16 = jnp.zeros((LANES,), jnp.float32)

    def zb(i, c):
        rows[i // (COLW // LANES),
             pl.ds((i % (COLW // LANES)) * LANES, LANES)] = zero16
        return c

    lax.fori_loop(0, RCH * (COLW // LANES), zb, 0)
    r0 = sid * zstripe
    for off, sz in _chunks(zstripe, RCH):
        pltpu.sync_copy(rows.at[pl.ds(0, sz)], acc.at[pl.ds(r0 + off, sz)])


def _seg_writeout(cid, sid, p, rows, acc, out_hbm, half):
    wstripe = half // NS
    w0 = sid * wstripe
    nbase = cid * half
    for off, sz in _chunks(wstripe, RCH):
        pltpu.sync_copy(acc.at[pl.ds(w0 + off, sz)], rows.at[pl.ds(0, sz)])
        pltpu.sync_copy(rows.at[pl.ds(0, sz)],
                        out_hbm.at[pl.ds(nbase + w0 + off, sz),
                                   pl.ds(p * COLW, COLW)])


def _seg_idx(dv, idx, nbase, half):
    for c8 in range(RCH // LANES):
        dd = dv[pl.ds(c8 * LANES, LANES)]
        lo = dd - nbase
        ok = (lo >= 0) & (lo < half)
        idx[0, pl.ds(c8 * LANES, LANES)] = jnp.where(
            ok, lo, half + (dd & (TB - 1)))


def _idx_precompute(idxv, nbase, half, nch):
    def row(r, c):
        for c8 in range(RCH // LANES):
            dd = idxv[r, pl.ds(c8 * LANES, LANES)]
            lo = dd - nbase
            ok = (lo >= 0) & (lo < half)
            idxv[r, pl.ds(c8 * LANES, LANES)] = jnp.where(
                ok, lo, half + (dd & (TB - 1)))
        return c

    lax.fori_loop(0, nch, row, 0)


def _segsum(msg, dst2d, n_pad, e_pad):
    half = n_pad // 2
    acc_rows = half + TB
    zstripe = acc_rows // NS
    tr = e_pad // NS
    nch = tr // RCH
    mesh = plsc.VectorSubcoreMesh(**_MESH)

    def body(msg_hbm, d_hbm, out_hbm, acc, bufa, bufb, idxv, sema, semb,
             sca, scb):
        cid = lax.axis_index("c")
        sid = lax.axis_index("s")
        ebase = sid * tr
        nbase = cid * half
        pltpu.sync_copy(d_hbm.at[pl.ds(sid * nch, nch)], idxv)
        _idx_precompute(idxv, nbase, half, nch)

        def rd(j, buf, sem):
            return pltpu.async_copy(
                msg_hbm.at[pl.ds(ebase + j * RCH, RCH),
                           pl.ds(p * COLW, COLW)], buf, sem)

        def wait_rd(buf, sem):
            pltpu.make_async_copy(msg_hbm.at[pl.ds(ebase, RCH),
                                             pl.ds(0, COLW)],
                                  buf, sem).wait()

        def wait_sc(buf, sem):
            pltpu.make_async_copy(buf, acc.at[idxv.at[0]], sem).wait()

        for p in range(HID // COLW):
            _seg_zero(sid, bufa, acc, zstripe)
            plsc.subcore_barrier()
            rd(0, bufb, semb)

            def pair(k, c):
                j0 = 2 * k
                j1 = j0 + 1
                j2 = jnp.minimum(j1 + 1, nch - 1)
                wait_rd(bufb, semb)

                @pl.when(k > 0)
                def _():
                    wait_sc(bufa, sca)

                rd(j1, bufa, sema)
                pltpu.async_copy(bufb, acc.at[idxv.at[j0]], scb, add=True)
                wait_rd(bufa, sema)
                pltpu.async_copy(bufa, acc.at[idxv.at[j1]], sca, add=True)
                wait_sc(bufb, scb)
                rd(j2, bufb, semb)
                return c

            lax.fori_loop(0, nch // 2, pair, 0)
            wait_rd(bufb, semb)
            wait_sc(bufa, sca)
            plsc.subcore_barrier()
            _seg_writeout(cid, sid, p, bufa, acc, out_hbm, half)
            plsc.subcore_barrier()

    k = pl.kernel(
        body,
        mesh=mesh,
        out_type=jax.ShapeDtypeStruct((n_pad, HID), jnp.float32),
        scratch_types=[
            pltpu.VMEM_SHARED((acc_rows, COLW), jnp.float32),
            pltpu.VMEM((RCH, COLW), jnp.float32),
            pltpu.VMEM((RCH, COLW), jnp.float32),
            pltpu.VMEM((tr // RCH, RCH), jnp.int32),
            pltpu.SemaphoreType.DMA,
            pltpu.SemaphoreType.DMA,
            pltpu.SemaphoreType.DMA,
            pltpu.SemaphoreType.DMA,
        ],
    )
    return k(msg, dst2d)


def _spmm(fs, src2d, dst2d, n_pad, e_pad):
    half = n_pad // 2
    acc_rows = half + TB
    zstripe = acc_rows // NS
    tr = e_pad // NS
    nch = tr // RCH
    mesh = plsc.VectorSubcoreMesh(**_MESH)

    def body(fs_hbm, s_hbm, d_hbm, out_hbm, acc, bufa, bufb, srcv, idxv,
             sema, semb, sca, scb):
        cid = lax.axis_index("c")
        sid = lax.axis_index("s")
        nbase = cid * half
        pltpu.sync_copy(s_hbm.at[pl.ds(sid * nch, nch)], srcv)
        pltpu.sync_copy(d_hbm.at[pl.ds(sid * nch, nch)], idxv)
        _idx_precompute(idxv, nbase, half, nch)

        def rd(j, buf, sem):
            return pltpu.async_copy(
                fs_hbm.at[srcv.at[j], pl.ds(p * COLW, COLW)], buf, sem)

        def wait_rd(buf, sem):
            pltpu.make_async_copy(fs_hbm.at[srcv.at[0], pl.ds(0, COLW)],
                                  buf, sem).wait()

        def wait_sc(buf, sem):
            pltpu.make_async_copy(buf, acc.at[idxv.at[0]], sem).wait()

        for p in range(HID // COLW):
            _seg_zero(sid, bufa, acc, zstripe)
            plsc.subcore_barrier()
            rd(0, bufb, semb)

            def pair(k, c):
                j0 = 2 * k
                j1 = j0 + 1
                j2 = jnp.minimum(j1 + 1, nch - 1)
                wait_rd(bufb, semb)

                @pl.when(k > 0)
                def _():
                    wait_sc(bufa, sca)

                rd(j1, bufa, sema)
                pltpu.async_copy(bufb, acc.at[idxv.at[j0]], scb, add=True)
                wait_rd(bufa, sema)
                pltpu.async_copy(bufa, acc.at[idxv.at[j1]], sca, add=True)
                wait_sc(bufb, scb)
                rd(j2, bufb, semb)
                return c

            lax.fori_loop(0, nch // 2, pair, 0)
            wait_rd(bufb, semb)
            wait_sc(bufa, sca)
            plsc.subcore_barrier()
            _seg_writeout(cid, sid, p, bufa, acc, out_hbm, half)
            plsc.subcore_barrier()

    k = pl.kernel(
        body,
        mesh=mesh,
        out_type=jax.ShapeDtypeStruct((n_pad, HID), jnp.float32),
        scratch_types=[
            pltpu.VMEM_SHARED((acc_rows, COLW), jnp.float32),
            pltpu.VMEM((RCH, COLW), jnp.float32),
            pltpu.VMEM((RCH, COLW), jnp.float32),
            pltpu.VMEM((tr // RCH, RCH), jnp.int32),
            pltpu.VMEM((tr // RCH, RCH), jnp.int32),
            pltpu.SemaphoreType.DMA,
            pltpu.SemaphoreType.DMA,
            pltpu.SemaphoreType.DMA,
            pltpu.SemaphoreType.DMA,
        ],
    )
    return k(fs, src2d, dst2d)


def _dot(a, b):
    return jnp.dot(a, b, preferred_element_type=jnp.float32)


def _deg_isqrt(od16, id16):
    n_pad = od16.shape[0]
    blk = n_pad // 8

    def body(od_r, id_r, os_r, oi_r):
        os_r[:] = lax.rsqrt(
            jnp.maximum(jnp.sum(od_r[:], axis=1, keepdims=True), 1.0))
        oi_r[:] = lax.rsqrt(
            jnp.maximum(jnp.sum(id_r[:], axis=1, keepdims=True), 1.0))

    return pl.pallas_call(
        body,
        grid=(n_pad // blk,),
        in_specs=[
            pl.BlockSpec((blk, COLW), lambda i: (i, 0)),
            pl.BlockSpec((blk, COLW), lambda i: (i, 0)),
        ],
        out_specs=[
            pl.BlockSpec((blk, 1), lambda i: (i, 0)),
            pl.BlockSpec((blk, 1), lambda i: (i, 0)),
        ],
        out_shape=[
            jax.ShapeDtypeStruct((n_pad, 1), jnp.float32),
            jax.ShapeDtypeStruct((n_pad, 1), jnp.float32),
        ],
    )(od16, id16)


def _edge_mlp(hs, hd, rad, ef, wsrc, wdst, wr, wef, b1, w2, b2, ncoord=0):
    e_pad, dd = hs.shape
    blk = 1024
    has_ef = ef is not None
    emit_rad = rad is None

    def body(*refs):
        if has_ef:
            (hs_r, hd_r, ef_r, wsrc_r, wdst_r, wr_r, wef_r, b1_r,
             w2_r, b2_r, out_r, rad_r) = refs
        else:
            (hs_r, hd_r, rad_r, wsrc_r, wdst_r, wr_r, b1_r, w2_r, b2_r,
             out_r) = refs
        if emit_rad:
            d = hs_r[:] - hd_r[:]
            cmask = (lax.broadcasted_iota(jnp.int32, (1, dd), 1)
                     < ncoord).astype(jnp.float32)
            radv = jnp.sum(d * d * cmask, axis=1, keepdims=True)
            rad_r[:] = radv
        else:
            radv = rad_r[:]
        f = _dot(hs_r[:], wsrc_r[:]) + _dot(hd_r[:], wdst_r[:])
        f = f + radv * wr_r[:] + b1_r[:]
        if has_ef:
            f = f + _dot(ef_r[:], wef_r[:])
        m = _silu(f)
        out_r[:] = _silu(_dot(m, w2_r[:]) + b2_r[:])

    in_specs = [
        pl.BlockSpec((blk, dd), lambda i: (i, 0)),
        pl.BlockSpec((blk, dd), lambda i: (i, 0)),
    ]
    args = [hs, hd]
    if not emit_rad:
        in_specs.append(pl.BlockSpec((blk, 1), lambda i: (i, 0)))
        args.append(rad)
    if has_ef:
        in_specs.append(pl.BlockSpec((blk, ef.shape[1]), lambda i: (i, 0)))
        args.append(ef)
    full = [wsrc, wdst, wr] + ([wef] if has_ef else []) + [b1, w2, b2]
    for w in full:
        in_specs.append(pl.BlockSpec(w.shape, lambda i: (0, 0)))
        args.append(w)
    out_specs = pl.BlockSpec((blk, HID), lambda i: (i, 0))
    out_shape = jax.ShapeDtypeStruct((e_pad, HID), jnp.float32)
    if emit_rad:
        out_specs = [out_specs, pl.BlockSpec((blk, 1), lambda i: (i, 0))]
        out_shape = [out_shape, jax.ShapeDtypeStruct((e_pad, 1), jnp.float32)]
    return pl.pallas_call(
        body,
        grid=(e_pad // blk,),
        in_specs=in_specs,
        out_specs=out_specs,
        out_shape=out_shape,
    )(*args)


def _node_mlp(h, hn, w1h, w1n, b1, w2, b2, oisq=None):
    n_pad, dh = h.shape
    blk = n_pad // 8
    scale = oisq is not None

    def body(*refs):
        if scale:
            h_r, hn_r, w1h_r, w1n_r, b1_r, w2_r, b2_r, oq_r, out_r, outs_r \
                = refs
        else:
            h_r, hn_r, w1h_r, w1n_r, b1_r, w2_r, b2_r, out_r = refs
        a = _silu(_dot(h_r[:], w1h_r[:]) + _dot(hn_r[:], w1n_r[:]) + b1_r[:])
        o = _dot(a, w2_r[:]) + b2_r[:]
        out_r[:] = o
        if scale:
            outs_r[:] = o * oq_r[:]

    in_specs = [
        pl.BlockSpec((blk, dh), lambda i: (i, 0)),
        pl.BlockSpec((blk, HID), lambda i: (i, 0)),
        pl.BlockSpec(w1h.shape, lambda i: (0, 0)),
        pl.BlockSpec(w1n.shape, lambda i: (0, 0)),
        pl.BlockSpec(b1.shape, lambda i: (0, 0)),
        pl.BlockSpec(w2.shape, lambda i: (0, 0)),
        pl.BlockSpec(b2.shape, lambda i: (0, 0)),
    ]
    args = [h, hn, w1h, w1n, b1, w2, b2]
    out_specs = pl.BlockSpec((blk, HID), lambda i: (i, 0))
    out_shape = jax.ShapeDtypeStruct((n_pad, HID), jnp.float32)
    if scale:
        in_specs.append(pl.BlockSpec((blk, 1), lambda i: (i, 0)))
        args.append(oisq)
        out_specs = [out_specs, pl.BlockSpec((blk, HID), lambda i: (i, 0))]
        out_shape = [out_shape,
                     jax.ShapeDtypeStruct((n_pad, HID), jnp.float32)]
    return pl.pallas_call(
        body,
        grid=(n_pad // blk,),
        in_specs=in_specs,
        out_specs=out_specs,
        out_shape=out_shape,
    )(*args)


def _gcn_update(agg, f0, iisq, oisq, w, b, beta):
    n_pad = agg.shape[0]
    blk = n_pad // 8

    def body(ag_r, f0_r, iq_r, oq_r, w_r, b_r, out_r, outs_r):
        g = (ag_r[:] * iq_r[:]) * (1.0 - ALPHA) + f0_r[:] * ALPHA
        rst = g * (1.0 - beta) + _dot(g, w_r[:]) * beta + b_r[:]
        o = jnp.where(rst > 0, rst, jnp.exp(rst) - 1.0)
        out_r[:] = o
        outs_r[:] = o * oq_r[:]

    return pl.pallas_call(
        body,
        grid=(n_pad // blk,),
        in_specs=[
            pl.BlockSpec((blk, HID), lambda i: (i, 0)),
            pl.BlockSpec((blk, HID), lambda i: (i, 0)),
            pl.BlockSpec((blk, 1), lambda i: (i, 0)),
            pl.BlockSpec((blk, 1), lambda i: (i, 0)),
            pl.BlockSpec(w.shape, lambda i: (0, 0)),
            pl.BlockSpec(b.shape, lambda i: (0, 0)),
        ],
        out_specs=[
            pl.BlockSpec((blk, HID), lambda i: (i, 0)),
            pl.BlockSpec((blk, HID), lambda i: (i, 0)),
        ],
        out_shape=[
            jax.ShapeDtypeStruct((n_pad, HID), jnp.float32),
            jax.ShapeDtypeStruct((n_pad, HID), jnp.float32),
        ],
    )(agg, f0, iisq, oisq, w, b)


def _ceil_to(x, m):
    return (x + m - 1) // m * m


def kernel(node_feats, coord, edge_feats, edge_index, params):
    n, din = node_feats.shape
    e = edge_index.shape[1]
    n_pad = _ceil_to(n, 2 * NS * RCH // 2)
    e_pad = _ceil_to(e, NW * CH)

    src = edge_index[0]
    dst = edge_index[1]
    pad_e = e_pad - e
    srcp = jnp.pad(src, (0, pad_e), constant_values=-1)
    dstp = jnp.pad(dst, (0, pad_e), constant_values=-1)
    srcg = jnp.maximum(srcp, 0)
    dstg = jnp.maximum(dstp, 0)
    src2d = srcg.reshape(e_pad // RCH, RCH)
    dst2d = dstp.reshape(e_pad // RCH, RCH)
    ones128 = jnp.zeros((CH, COLW), jnp.float32).at[:, 0].set(1.0)
    od, idg = _deg_ones(srcp, dstp, ones128, n_pad, e_pad)
    oisq, iisq = _deg_isqrt(od, idg)
    efp = jnp.pad(edge_feats, ((0, pad_e), (0, 0)))

    ncoord = coord.shape[1]
    din_p = _ceil_to(din + ncoord, 128)
    h = jnp.pad(jnp.concatenate([coord, node_feats], axis=1),
                ((0, n_pad - n), (0, din_p - din - ncoord)))

    p0 = params["egnn0"]
    zc = jnp.zeros((ncoord, HID), jnp.float32)
    w1 = p0["edge1"]["W"]
    wsrc = jnp.pad(jnp.concatenate([zc, w1[:din]], axis=0),
                   ((0, din_p - din - ncoord), (0, 0)))
    wdst = jnp.pad(jnp.concatenate([zc, w1[din:2 * din]], axis=0),
                   ((0, din_p - din - ncoord), (0, 0)))
    wr = w1[2 * din:2 * din + 1]
    wef = w1[2 * din + 1:]
    hs, hd = _gather2(h, srcg, dstg, e_pad)
    msg, rad = _edge_mlp(hs, hd, None, efp, wsrc, wdst, wr, wef,
                         p0["edge1"]["b"][None], p0["edge2"]["W"],
                         p0["edge2"]["b"][None], ncoord=ncoord)
    hn = _segsum(msg, dst2d, n_pad, e_pad)
    n1 = p0["node1"]["W"]
    w1h = jnp.pad(jnp.concatenate([zc, n1[:din]], axis=0),
                  ((0, din_p - din - ncoord), (0, 0)))
    h = _node_mlp(h, hn, w1h, n1[din:], p0["node1"]["b"][None],
                  p0["node2"]["W"], p0["node2"]["b"][None])
    h0 = h

    n_extra = len(params["egnn_layers"])
    feat_s = None
    for li, p in enumerate(params["egnn_layers"]):
        w1 = p["edge1"]["W"]
        hs, hd = _gather2(h, srcg, dstg, e_pad)
        msg = _edge_mlp(hs, hd, rad, None, w1[:HID], w1[HID:2 * HID],
                        w1[2 * HID:], None, p["edge1"]["b"][None],
                        p["edge2"]["W"], p["edge2"]["b"][None])
        hn = _segsum(msg, dst2d, n_pad, e_pad)
        n1 = p["node1"]["W"]
        last = li == n_extra - 1
        res = _node_mlp(h, hn, n1[:HID], n1[HID:], p["node1"]["b"][None],
                        p["node2"]["W"], p["node2"]["b"][None],
                        oisq=oisq if last else None)
        if last:
            h, feat_s = res
        else:
            h = res

    feat = h
    for i, p in enumerate(params["gcn_layers"]):
        beta = float(np.log(1.0 / (i + 1) + 1.0))
        agg = _spmm(feat_s, src2d, dst2d, n_pad, e_pad)
        feat, feat_s = _gcn_update(agg, h0, iisq, oisq, p["W"], p["b"][None],
                                   beta)
    return feat[:n]


# speedup vs baseline: 1.2636x; 1.2636x over previous
import functools

import numpy as np
import jax
import jax.numpy as jnp
from jax import lax
from jax.experimental import pallas as pl
from jax.experimental.pallas import tpu as pltpu
from jax.experimental.pallas import tpu_sc as plsc

NC = 2
NS = 16
NW = NC * NS
LANES = 16
CH = 128
RCH = 128
TB = 128
COLW = 128
HID = 256
ALPHA = 0.5

_MESH = dict(core_axis_name="c", subcore_axis_name="s",
             num_cores=NC, num_subcores=NS)


def _silu(x):
    return x * jax.nn.sigmoid(x)


def _chunks(total, step):
    out = []
    off = 0
    while off < total:
        sz = min(step, total - off)
        out.append((off, sz))
        off += sz
    return out


def _deg_ones(srcp, dstp, ones128, n_pad, e_pad):
    half = n_pad // 2
    acc_rows = half + TB
    zstripe = acc_rows // NS
    tr = e_pad // NS
    nch = tr // RCH
    mesh = plsc.VectorSubcoreMesh(**_MESH)

    def body(s_hbm, d_hbm, ones_hbm, od_hbm, id_hbm,
             acco, acci, buf, sv, dv, idxs, idxd):
        cid = lax.axis_index("c")
        sid = lax.axis_index("s")
        _seg_zero(sid, buf, acco, zstripe)
        r0 = sid * zstripe
        for off, sz in _chunks(zstripe, RCH):
            pltpu.sync_copy(buf.at[pl.ds(0, sz)], acci.at[pl.ds(r0 + off, sz)])
        pltpu.sync_copy(ones_hbm, buf)
        ebase = sid * tr
        plsc.subcore_barrier()
        nbase = cid * half

        def step(j, c):
            off = j * RCH
            pltpu.sync_copy(s_hbm.at[pl.ds(ebase + off, RCH)], sv)
            pltpu.sync_copy(d_hbm.at[pl.ds(ebase + off, RCH)], dv)
            _seg_idx(sv, idxs, nbase, half)
            _seg_idx(dv, idxd, nbase, half)
            pltpu.sync_copy(buf, acco.at[idxs.at[0]], add=True)
            pltpu.sync_copy(buf, acci.at[idxd.at[0]], add=True)
            return c

        lax.fori_loop(0, nch, step, 0)
        plsc.subcore_barrier()
        _seg_writeout(cid, sid, 0, buf, acco, od_hbm, half)
        _seg_writeout(cid, sid, 0, buf, acci, id_hbm, half)

    k = pl.kernel(
        body,
        mesh=mesh,
        out_type=(
            jax.ShapeDtypeStruct((n_pad, COLW), jnp.float32),
            jax.ShapeDtypeStruct((n_pad, COLW), jnp.float32),
        ),
        scratch_types=[
            pltpu.VMEM_SHARED((acc_rows, COLW), jnp.float32),
            pltpu.VMEM_SHARED((acc_rows, COLW), jnp.float32),
            pltpu.VMEM((RCH, COLW), jnp.float32),
            pltpu.VMEM((RCH,), jnp.int32),
            pltpu.VMEM((RCH,), jnp.int32),
            pltpu.VMEM((1, RCH), jnp.int32),
            pltpu.VMEM((1, RCH), jnp.int32),
        ],
    )
    return k(srcp, dstp, ones128)


def _gather2(h2, srcg, dstg, e_pad):
    d = h2.shape[1]
    tr = e_pad // NW
    nch = tr // CH
    mesh = plsc.VectorSubcoreMesh(**_MESH)

    def body(h_hbm, s_hbm, d_hbm, hs_hbm, hd_hbm, siv, div, bufa, bufb,
             sema, semb):
        cid = lax.axis_index("c")
        sid = lax.axis_index("s")
        wid = sid * NC + cid
        base = wid * tr
        pltpu.sync_copy(s_hbm.at[pl.ds(base, tr)], siv)
        pltpu.sync_copy(d_hbm.at[pl.ds(base, tr)], div)

        def step(j, c):
            off = j * CH
            cpa = pltpu.async_copy(h_hbm.at[siv.at[pl.ds(off, CH)]], bufa,
                                   sema)
            cpb = pltpu.async_copy(h_hbm.at[div.at[pl.ds(off, CH)]], bufb,
                                   semb)
            cpa.wait()
            pltpu.sync_copy(bufa, hs_hbm.at[pl.ds(base + off, CH)])
            cpb.wait()
            pltpu.sync_copy(bufb, hd_hbm.at[pl.ds(base + off, CH)])
            return c

        lax.fori_loop(0, nch, step, 0)

    k = pl.kernel(
        body,
        mesh=mesh,
        out_type=(
            jax.ShapeDtypeStruct((e_pad, d), jnp.float32),
            jax.ShapeDtypeStruct((e_pad, d), jnp.float32),
        ),
        scratch_types=[
            pltpu.VMEM((tr,), jnp.int32),
            pltpu.VMEM((tr,), jnp.int32),
            pltpu.VMEM((CH, d), jnp.float32),
            pltpu.VMEM((CH, d), jnp.float32),
            pltpu.SemaphoreType.DMA,
            pltpu.SemaphoreType.DMA,
        ],
    )
    return k(h2, srcg, dstg)


def _seg_zero(sid, rows, acc, zstripe):
    zero16 = jnp.zeros((LANES,), jnp.float32)

    def zb(i, c):
        rows[i // (COLW // LANES),
             pl.ds((i % (COLW // LANES)) * LANES, LANES)] = zero16
        return c

    lax.fori_loop(0, RCH * (COLW // LANES), zb, 0)
    r0 = sid * zstripe
    for off, sz in _chunks(zstripe, RCH):
        pltpu.sync_copy(rows.at[pl.ds(0, sz)], acc.at[pl.ds(r0 + off, sz)])


def _seg_writeout(cid, sid, p, rows, acc, out_hbm, half):
    wstripe = half // NS
    w0 = sid * wstripe
    nbase = cid * half
    for off, sz in _chunks(wstripe, RCH):
        pltpu.sync_copy(acc.at[pl.ds(w0 + off, sz)], rows.at[pl.ds(0, sz)])
        pltpu.sync_copy(rows.at[pl.ds(0, sz)],
                        out_hbm.at[pl.ds(nbase + w0 + off, sz),
                                   pl.ds(p * COLW, COLW)])


def _seg_idx(dv, idx, nbase, half):
    for c8 in range(RCH // LANES):
        dd = dv[pl.ds(c8 * LANES, LANES)]
        lo = dd - nbase
        ok = (lo >= 0) & (lo < half)
        idx[0, pl.ds(c8 * LANES, LANES)] = jnp.where(
            ok, lo, half + (dd & (TB - 1)))


def _idx_precompute(idxv, nbase, half, nch):
    def row(r, c):
        for c8 in range(RCH // LANES):
            dd = idxv[r, pl.ds(c8 * LANES, LANES)]
            lo = dd - nbase
            ok = (lo >= 0) & (lo < half)
            idxv[r, pl.ds(c8 * LANES, LANES)] = jnp.where(
                ok, lo, half + (dd & (TB - 1)))
        return c

    lax.fori_loop(0, nch, row, 0)


def _idx_prep(idxv, n, nch):
    iota = lax.iota(jnp.int32, LANES)

    def row(r, c):
        for c8 in range(RCH // LANES):
            dd = idxv[r, pl.ds(c8 * LANES, LANES)]
            trash = n + ((r * RCH + c8 * LANES + iota) & 63)
            idxv[r, pl.ds(c8 * LANES, LANES)] = jnp.where(dd >= 0, dd, trash)
        return c

    lax.fori_loop(0, nch, row, 0)


def _seg_kernel(gather_mode, n, n_pad, e_pad):
    half_e = e_pad // 2
    tr = half_e // NS
    nch = tr // RCH
    acc_rows = _ceil_to(n + 64, RCH)
    zstripe = acc_rows // NS
    mesh = plsc.VectorSubcoreMesh(**_MESH)

    def body(*refs):
        if gather_mode:
            (tab_hbm, s_hbm, d_hbm, out_hbm, acc, bufa, bufb, srcv, idxv,
             sema, semb, sca, scb) = refs
        else:
            (tab_hbm, d_hbm, out_hbm, acc, bufa, bufb, idxv,
             sema, semb, sca, scb) = refs
        cid = lax.axis_index("c")
        sid = lax.axis_index("s")
        ebase = cid * half_e + sid * tr
        rbase = cid * (half_e // RCH) + sid * nch
        pltpu.sync_copy(d_hbm.at[pl.ds(rbase, nch)], idxv)
        _idx_prep(idxv, n, nch)
        if gather_mode:
            pltpu.sync_copy(s_hbm.at[pl.ds(rbase, nch)], srcv)

            def rd(j, buf, sem):
                return pltpu.async_copy(
                    tab_hbm.at[srcv.at[j], pl.ds(p * COLW, COLW)], buf, sem)

            def wait_rd(buf, sem):
                pltpu.make_async_copy(
                    tab_hbm.at[srcv.at[0], pl.ds(0, COLW)], buf, sem).wait()
        else:

            def rd(j, buf, sem):
                return pltpu.async_copy(
                    tab_hbm.at[pl.ds(ebase + j * RCH, RCH),
                               pl.ds(p * COLW, COLW)], buf, sem)

            def wait_rd(buf, sem):
                pltpu.make_async_copy(
                    tab_hbm.at[pl.ds(ebase, RCH), pl.ds(0, COLW)],
                    buf, sem).wait()

        def wait_sc(buf, sem):
            pltpu.make_async_copy(buf, acc.at[idxv.at[0]], sem).wait()

        for p in range(HID // COLW):
            _seg_zero(sid, bufa, acc, zstripe)
            plsc.subcore_barrier()
            rd(0, bufb, semb)

            def pair(k, c):
                j0 = 2 * k
                j1 = j0 + 1
                j2 = jnp.minimum(j1 + 1, nch - 1)
                wait_rd(bufb, semb)

                @pl.when(k > 0)
                def _():
                    wait_sc(bufa, sca)

                rd(j1, bufa, sema)
                pltpu.async_copy(bufb, acc.at[idxv.at[j0]], scb, add=True)
                wait_rd(bufa, sema)
                pltpu.async_copy(bufa, acc.at[idxv.at[j1]], sca, add=True)
                wait_sc(bufb, scb)
                rd(j2, bufb, semb)
                return c

            lax.fori_loop(0, nch // 2, pair, 0)
            wait_rd(bufb, semb)
            wait_sc(bufa, sca)
            plsc.subcore_barrier()
            w0 = sid * zstripe
            obase = cid * n_pad
            for off, sz in _chunks(zstripe, RCH):
                pltpu.sync_copy(acc.at[pl.ds(w0 + off, sz)],
                                bufa.at[pl.ds(0, sz)])
                pltpu.sync_copy(bufa.at[pl.ds(0, sz)],
                                out_hbm.at[pl.ds(obase + w0 + off, sz),
                                           pl.ds(p * COLW, COLW)])
            plsc.subcore_barrier()

    scratch = [
        pltpu.VMEM_SHARED((acc_rows, COLW), jnp.float32),
        pltpu.VMEM((RCH, COLW), jnp.float32),
        pltpu.VMEM((RCH, COLW), jnp.float32),
    ]
    if gather_mode:
        scratch.append(pltpu.VMEM((nch, RCH), jnp.int32))
    scratch += [
        pltpu.VMEM((nch, RCH), jnp.int32),
        pltpu.SemaphoreType.DMA,
        pltpu.SemaphoreType.DMA,
        pltpu.SemaphoreType.DMA,
        pltpu.SemaphoreType.DMA,
    ]
    k = pl.kernel(
        body,
        mesh=mesh,
        out_type=jax.ShapeDtypeStruct((2 * n_pad, HID), jnp.float32),
        scratch_types=scratch,
    )
    return k


def _segsum(msg, dst2d, n, n_pad, e_pad):
    r = _seg_kernel(False, n, n_pad, e_pad)(msg, dst2d)
    return r[:n_pad], r[n_pad:]


def _spmm(fs, src2d, dst2d, n, n_pad, e_pad):
    r = _seg_kernel(True, n, n_pad, e_pad)(fs, src2d, dst2d)
    return r[:n_pad], r[n_pad:]


def _dot(a, b):
    return jnp.dot(a, b, preferred_element_type=jnp.float32)


def _deg_isqrt(od16, id16):
    n_pad = od16.shape[0]
    blk = n_pad // 8

    def body(od_r, id_r, os_r, oi_r):
        os_r[:] = lax.rsqrt(
            jnp.maximum(jnp.sum(od_r[:], axis=1, keepdims=True), 1.0))
        oi_r[:] = lax.rsqrt(
            jnp.maximum(jnp.sum(id_r[:], axis=1, keepdims=True), 1.0))

    return pl.pallas_call(
        body,
        grid=(n_pad // blk,),
        in_specs=[
            pl.BlockSpec((blk, COLW), lambda i: (i, 0)),
            pl.BlockSpec((blk, COLW), lambda i: (i, 0)),
        ],
        out_specs=[
            pl.BlockSpec((blk, 1), lambda i: (i, 0)),
            pl.BlockSpec((blk, 1), lambda i: (i, 0)),
        ],
        out_shape=[
            jax.ShapeDtypeStruct((n_pad, 1), jnp.float32),
            jax.ShapeDtypeStruct((n_pad, 1), jnp.float32),
        ],
    )(od16, id16)


def _edge_mlp(hs, hd, rad, ef, wsrc, wdst, wr, wef, b1, w2, b2, ncoord=0):
    e_pad, dd = hs.shape
    blk = 1024
    has_ef = ef is not None
    emit_rad = rad is None

    def body(*refs):
        if has_ef:
            (hs_r, hd_r, ef_r, wsrc_r, wdst_r, wr_r, wef_r, b1_r,
             w2_r, b2_r, out_r, rad_r) = refs
        else:
            (hs_r, hd_r, rad_r, wsrc_r, wdst_r, wr_r, b1_r, w2_r, b2_r,
             out_r) = refs
        if emit_rad:
            d = hs_r[:] - hd_r[:]
            cmask = (lax.broadcasted_iota(jnp.int32, (1, dd), 1)
                     < ncoord).astype(jnp.float32)
            radv = jnp.sum(d * d * cmask, axis=1, keepdims=True)
            rad_r[:] = radv
        else:
            radv = rad_r[:]
        f = _dot(hs_r[:], wsrc_r[:]) + _dot(hd_r[:], wdst_r[:])
        f = f + radv * wr_r[:] + b1_r[:]
        if has_ef:
            f = f + _dot(ef_r[:], wef_r[:])
        m = _silu(f)
        out_r[:] = _silu(_dot(m, w2_r[:]) + b2_r[:])

    in_specs = [
        pl.BlockSpec((blk, dd), lambda i: (i, 0)),
        pl.BlockSpec((blk, dd), lambda i: (i, 0)),
    ]
    args = [hs, hd]
    if not emit_rad:
        in_specs.append(pl.BlockSpec((blk, 1), lambda i: (i, 0)))
        args.append(rad)
    if has_ef:
        in_specs.append(pl.BlockSpec((blk, ef.shape[1]), lambda i: (i, 0)))
        args.append(ef)
    full = [wsrc, wdst, wr] + ([wef] if has_ef else []) + [b1, w2, b2]
    for w in full:
        in_specs.append(pl.BlockSpec(w.shape, lambda i: (0, 0)))
        args.append(w)
    out_specs = pl.BlockSpec((blk, HID), lambda i: (i, 0))
    out_shape = jax.ShapeDtypeStruct((e_pad, HID), jnp.float32)
    if emit_rad:
        out_specs = [out_specs, pl.BlockSpec((blk, 1), lambda i: (i, 0))]
        out_shape = [out_shape, jax.ShapeDtypeStruct((e_pad, 1), jnp.float32)]
    return pl.pallas_call(
        body,
        grid=(e_pad // blk,),
        in_specs=in_specs,
        out_specs=out_specs,
        out_shape=out_shape,
    )(*args)


def _node_mlp(h, hn0, hn1, w1h, w1n, b1, w2, b2, oisq=None):
    n_pad, dh = h.shape
    blk = n_pad // 8
    scale = oisq is not None

    def body(*refs):
        if scale:
            (h_r, hn0_r, hn1_r, w1h_r, w1n_r, b1_r, w2_r, b2_r, oq_r, out_r,
             outs_r) = refs
        else:
            h_r, hn0_r, hn1_r, w1h_r, w1n_r, b1_r, w2_r, b2_r, out_r = refs
        hn_r = hn0_r[:] + hn1_r[:]
        a = _silu(_dot(h_r[:], w1h_r[:]) + _dot(hn_r, w1n_r[:]) + b1_r[:])
        o = _dot(a, w2_r[:]) + b2_r[:]
        out_r[:] = o
        if scale:
            outs_r[:] = o * oq_r[:]

    in_specs = [
        pl.BlockSpec((blk, dh), lambda i: (i, 0)),
        pl.BlockSpec((blk, HID), lambda i: (i, 0)),
        pl.BlockSpec((blk, HID), lambda i: (i, 0)),
        pl.BlockSpec(w1h.shape, lambda i: (0, 0)),
        pl.BlockSpec(w1n.shape, lambda i: (0, 0)),
        pl.BlockSpec(b1.shape, lambda i: (0, 0)),
        pl.BlockSpec(w2.shape, lambda i: (0, 0)),
        pl.BlockSpec(b2.shape, lambda i: (0, 0)),
    ]
    args = [h, hn0, hn1, w1h, w1n, b1, w2, b2]
    out_specs = pl.BlockSpec((blk, HID), lambda i: (i, 0))
    out_shape = jax.ShapeDtypeStruct((n_pad, HID), jnp.float32)
    if scale:
        in_specs.append(pl.BlockSpec((blk, 1), lambda i: (i, 0)))
        args.append(oisq)
        out_specs = [out_specs, pl.BlockSpec((blk, HID), lambda i: (i, 0))]
        out_shape = [out_shape,
                     jax.ShapeDtypeStruct((n_pad, HID), jnp.float32)]
    return pl.pallas_call(
        body,
        grid=(n_pad // blk,),
        in_specs=in_specs,
        out_specs=out_specs,
        out_shape=out_shape,
    )(*args)


def _gcn_update(agg0, agg1, f0, iisq, oisq, w, b, beta):
    n_pad = agg0.shape[0]
    blk = n_pad // 8

    def body(ag0_r, ag1_r, f0_r, iq_r, oq_r, w_r, b_r, out_r, outs_r):
        g = ((ag0_r[:] + ag1_r[:]) * iq_r[:]) * (1.0 - ALPHA) + f0_r[:] * ALPHA
        rst = g * (1.0 - beta) + _dot(g, w_r[:]) * beta + b_r[:]
        o = jnp.where(rst > 0, rst, jnp.exp(rst) - 1.0)
        out_r[:] = o
        outs_r[:] = o * oq_r[:]

    return pl.pallas_call(
        body,
        grid=(n_pad // blk,),
        in_specs=[
            pl.BlockSpec((blk, HID), lambda i: (i, 0)),
            pl.BlockSpec((blk, HID), lambda i: (i, 0)),
            pl.BlockSpec((blk, HID), lambda i: (i, 0)),
            pl.BlockSpec((blk, 1), lambda i: (i, 0)),
            pl.BlockSpec((blk, 1), lambda i: (i, 0)),
            pl.BlockSpec(w.shape, lambda i: (0, 0)),
            pl.BlockSpec(b.shape, lambda i: (0, 0)),
        ],
        out_specs=[
            pl.BlockSpec((blk, HID), lambda i: (i, 0)),
            pl.BlockSpec((blk, HID), lambda i: (i, 0)),
        ],
        out_shape=[
            jax.ShapeDtypeStruct((n_pad, HID), jnp.float32),
            jax.ShapeDtypeStruct((n_pad, HID), jnp.float32),
        ],
    )(agg0, agg1, f0, iisq, oisq, w, b)


def _ceil_to(x, m):
    return (x + m - 1) // m * m


def kernel(node_feats, coord, edge_feats, edge_index, params):
    n, din = node_feats.shape
    e = edge_index.shape[1]
    n_pad = _ceil_to(n, 2 * NS * RCH // 2)
    e_pad = _ceil_to(e, NW * CH)

    src = edge_index[0]
    dst = edge_index[1]
    pad_e = e_pad - e
    srcp = jnp.pad(src, (0, pad_e), constant_values=-1)
    dstp = jnp.pad(dst, (0, pad_e), constant_values=-1)
    srcg = jnp.maximum(srcp, 0)
    dstg = jnp.maximum(dstp, 0)
    src2d = srcg.reshape(e_pad // RCH, RCH)
    dst2d = dstp.reshape(e_pad // RCH, RCH)
    ones128 = jnp.zeros((CH, COLW), jnp.float32).at[:, 0].set(1.0)
    od, idg = _deg_ones(srcp, dstp, ones128, n_pad, e_pad)
    oisq, iisq = _deg_isqrt(od, idg)
    efp = jnp.pad(edge_feats, ((0, pad_e), (0, 0)))

    ncoord = coord.shape[1]
    din_p = _ceil_to(din + ncoord, 128)
    h = jnp.pad(jnp.concatenate([coord, node_feats], axis=1),
                ((0, n_pad - n), (0, din_p - din - ncoord)))

    p0 = params["egnn0"]
    zc = jnp.zeros((ncoord, HID), jnp.float32)
    w1 = p0["edge1"]["W"]
    wsrc = jnp.pad(jnp.concatenate([zc, w1[:din]], axis=0),
                   ((0, din_p - din - ncoord), (0, 0)))
    wdst = jnp.pad(jnp.concatenate([zc, w1[din:2 * din]], axis=0),
                   ((0, din_p - din - ncoord), (0, 0)))
    wr = w1[2 * din:2 * din + 1]
    wef = w1[2 * din + 1:]
    hs, hd = _gather2(h, srcg, dstg, e_pad)
    msg, rad = _edge_mlp(hs, hd, None, efp, wsrc, wdst, wr, wef,
                         p0["edge1"]["b"][None], p0["edge2"]["W"],
                         p0["edge2"]["b"][None], ncoord=ncoord)
    hn0, hn1 = _segsum(msg, dst2d, n, n_pad, e_pad)
    n1 = p0["node1"]["W"]
    w1h = jnp.pad(jnp.concatenate([zc, n1[:din]], axis=0),
                  ((0, din_p - din - ncoord), (0, 0)))
    h = _node_mlp(h, hn0, hn1, w1h, n1[din:], p0["node1"]["b"][None],
                  p0["node2"]["W"], p0["node2"]["b"][None])
    h0 = h

    n_extra = len(params["egnn_layers"])
    feat_s = None
    for li, p in enumerate(params["egnn_layers"]):
        w1 = p["edge1"]["W"]
        hs, hd = _gather2(h, srcg, dstg, e_pad)
        msg = _edge_mlp(hs, hd, rad, None, w1[:HID], w1[HID:2 * HID],
                        w1[2 * HID:], None, p["edge1"]["b"][None],
                        p["edge2"]["W"], p["edge2"]["b"][None])
        hn0, hn1 = _segsum(msg, dst2d, n, n_pad, e_pad)
        n1 = p["node1"]["W"]
        last = li == n_extra - 1
        res = _node_mlp(h, hn0, hn1, n1[:HID], n1[HID:],
                        p["node1"]["b"][None], p["node2"]["W"],
                        p["node2"]["b"][None], oisq=oisq if last else None)
        if last:
            h, feat_s = res
        else:
            h = res

    feat = h
    for i, p in enumerate(params["gcn_layers"]):
        beta = float(np.log(1.0 / (i + 1) + 1.0))
        agg0, agg1 = _spmm(feat_s, src2d, dst2d, n, n_pad, e_pad)
        feat, feat_s = _gcn_update(agg0, agg1, h0, iisq, oisq, p["W"], p["b"][None],
                                   beta)
    return feat[:n]


# speedup vs baseline: 1.3202x; 1.0448x over previous
import functools

import numpy as np
import jax
import jax.numpy as jnp
from jax import lax
from jax.experimental import pallas as pl
from jax.experimental.pallas import tpu as pltpu
from jax.experimental.pallas import tpu_sc as plsc

NC = 2
NS = 16
NW = NC * NS
LANES = 16
CH = 128
RCH = 128
TB = 128
COLW = 128
HID = 256
ALPHA = 0.5

_MESH = dict(core_axis_name="c", subcore_axis_name="s",
             num_cores=NC, num_subcores=NS)


def _silu(x):
    return x * jax.nn.sigmoid(x)


def _chunks(total, step):
    out = []
    off = 0
    while off < total:
        sz = min(step, total - off)
        out.append((off, sz))
        off += sz
    return out


def _deg_partial(ids2d, ones128, n, n_pad, e_pad):
    half_e = e_pad // 2
    tr = half_e // NS
    nch = tr // RCH
    acc_rows = _ceil_to(n + 64, RCH)
    zstripe = acc_rows // NS
    mesh = plsc.VectorSubcoreMesh(**_MESH)

    def body(d_hbm, ones_hbm, out_hbm, acc, buf, idxv, sc):
        cid = lax.axis_index("c")
        sid = lax.axis_index("s")
        rbase = cid * (half_e // RCH) + sid * nch
        pltpu.sync_copy(d_hbm.at[pl.ds(rbase, nch)], idxv)
        _idx_prep(idxv, n, nch)
        _seg_zero(sid, buf, acc, zstripe)
        pltpu.sync_copy(ones_hbm, buf)
        plsc.subcore_barrier()

        def step(j, c):
            pltpu.async_copy(buf, acc.at[idxv.at[j]], sc, add=True)
            return c

        lax.fori_loop(0, nch, step, 0)

        def drain(j, c):
            pltpu.make_async_copy(buf, acc.at[idxv.at[0]], sc).wait()
            return c

        lax.fori_loop(0, nch, drain, 0)
        plsc.subcore_barrier()
        w0 = sid * zstripe
        obase = cid * n_pad
        for off, sz in _chunks(zstripe, RCH):
            pltpu.sync_copy(acc.at[pl.ds(w0 + off, sz)],
                            buf.at[pl.ds(0, sz)])
            pltpu.sync_copy(buf.at[pl.ds(0, sz)],
                            out_hbm.at[pl.ds(obase + w0 + off, sz)])

    k = pl.kernel(
        body,
        mesh=mesh,
        out_type=jax.ShapeDtypeStruct((2 * n_pad, COLW), jnp.float32),
        scratch_types=[
            pltpu.VMEM_SHARED((acc_rows, COLW), jnp.float32),
            pltpu.VMEM((RCH, COLW), jnp.float32),
            pltpu.VMEM((nch, RCH), jnp.int32),
            pltpu.SemaphoreType.DMA,
        ],
    )
    return k(ids2d, ones128)


def _gather2(h2, srcg, dstg, e_pad):
    d = h2.shape[1]
    tr = e_pad // NW
    nch = tr // CH
    mesh = plsc.VectorSubcoreMesh(**_MESH)

    def body(h_hbm, s_hbm, d_hbm, hs_hbm, hd_hbm, siv, div, bufa, bufb,
             sema, semb):
        cid = lax.axis_index("c")
        sid = lax.axis_index("s")
        wid = sid * NC + cid
        base = wid * tr
        pltpu.sync_copy(s_hbm.at[pl.ds(base, tr)], siv)
        pltpu.sync_copy(d_hbm.at[pl.ds(base, tr)], div)

        def step(j, c):
            off = j * CH
            cpa = pltpu.async_copy(h_hbm.at[siv.at[pl.ds(off, CH)]], bufa,
                                   sema)
            cpb = pltpu.async_copy(h_hbm.at[div.at[pl.ds(off, CH)]], bufb,
                                   semb)
            cpa.wait()
            pltpu.sync_copy(bufa, hs_hbm.at[pl.ds(base + off, CH)])
            cpb.wait()
            pltpu.sync_copy(bufb, hd_hbm.at[pl.ds(base + off, CH)])
            return c

        lax.fori_loop(0, nch, step, 0)

    k = pl.kernel(
        body,
        mesh=mesh,
        out_type=(
            jax.ShapeDtypeStruct((e_pad, d), jnp.float32),
            jax.ShapeDtypeStruct((e_pad, d), jnp.float32),
        ),
        scratch_types=[
            pltpu.VMEM((tr,), jnp.int32),
            pltpu.VMEM((tr,), jnp.int32),
            pltpu.VMEM((CH, d), jnp.float32),
            pltpu.VMEM((CH, d), jnp.float32),
            pltpu.SemaphoreType.DMA,
            pltpu.SemaphoreType.DMA,
        ],
    )
    return k(h2, srcg, dstg)


def _seg_zero(sid, rows, acc, zstripe):
    zero16 = jnp.zeros((LANES,), jnp.float32)

    def zb(i, c):
        rows[i // (COLW // LANES),
             pl.ds((i % (COLW // LANES)) * LANES, LANES)] = zero16
        return c

    lax.fori_loop(0, RCH * (COLW // LANES), zb, 0)
    r0 = sid * zstripe
    for off, sz in _chunks(zstripe, RCH):
        pltpu.sync_copy(rows.at[pl.ds(0, sz)], acc.at[pl.ds(r0 + off, sz)])


def _seg_writeout(cid, sid, p, rows, acc, out_hbm, half):
    wstripe = half // NS
    w0 = sid * wstripe
    nbase = cid * half
    for off, sz in _chunks(wstripe, RCH):
        pltpu.sync_copy(acc.at[pl.ds(w0 + off, sz)], rows.at[pl.ds(0, sz)])
        pltpu.sync_copy(rows.at[pl.ds(0, sz)],
                        out_hbm.at[pl.ds(nbase + w0 + off, sz),
                                   pl.ds(p * COLW, COLW)])


def _seg_idx(dv, idx, nbase, half):
    for c8 in range(RCH // LANES):
        dd = dv[pl.ds(c8 * LANES, LANES)]
        lo = dd - nbase
        ok = (lo >= 0) & (lo < half)
        idx[0, pl.ds(c8 * LANES, LANES)] = jnp.where(
            ok, lo, half + (dd & (TB - 1)))


def _idx_precompute(idxv, nbase, half, nch):
    def row(r, c):
        for c8 in range(RCH // LANES):
            dd = idxv[r, pl.ds(c8 * LANES, LANES)]
            lo = dd - nbase
            ok = (lo >= 0) & (lo < half)
            idxv[r, pl.ds(c8 * LANES, LANES)] = jnp.where(
                ok, lo, half + (dd & (TB - 1)))
        return c

    lax.fori_loop(0, nch, row, 0)


def _idx_prep(idxv, n, nch):
    iota = lax.iota(jnp.int32, LANES)

    def row(r, c):
        for c8 in range(RCH // LANES):
            dd = idxv[r, pl.ds(c8 * LANES, LANES)]
            trash = n + ((r * RCH + c8 * LANES + iota) & 63)
            idxv[r, pl.ds(c8 * LANES, LANES)] = jnp.where(dd >= 0, dd, trash)
        return c

    lax.fori_loop(0, nch, row, 0)


def _seg_kernel(gather_mode, n, n_pad, e_pad):
    half_e = e_pad // 2
    tr = half_e // NS
    nch = tr // RCH
    acc_rows = _ceil_to(n + 64, RCH)
    zstripe = acc_rows // NS
    mesh = plsc.VectorSubcoreMesh(**_MESH)

    def body(*refs):
        if gather_mode:
            (tab_hbm, s_hbm, d_hbm, out_hbm, acc, bufa, bufb, srcv, idxv,
             sema, semb, sca, scb) = refs
        else:
            (tab_hbm, d_hbm, out_hbm, acc, bufa, bufb, idxv,
             sema, semb, sca, scb) = refs
        cid = lax.axis_index("c")
        sid = lax.axis_index("s")
        ebase = cid * half_e + sid * tr
        rbase = cid * (half_e // RCH) + sid * nch
        pltpu.sync_copy(d_hbm.at[pl.ds(rbase, nch)], idxv)
        _idx_prep(idxv, n, nch)
        if gather_mode:
            pltpu.sync_copy(s_hbm.at[pl.ds(rbase, nch)], srcv)

            def rd(j, buf, sem):
                return pltpu.async_copy(
                    tab_hbm.at[srcv.at[j], pl.ds(p * COLW, COLW)], buf, sem)

            def wait_rd(buf, sem):
                pltpu.make_async_copy(
                    tab_hbm.at[srcv.at[0], pl.ds(0, COLW)], buf, sem).wait()
        else:

            def rd(j, buf, sem):
                return pltpu.async_copy(
                    tab_hbm.at[pl.ds(ebase + j * RCH, RCH),
                               pl.ds(p * COLW, COLW)], buf, sem)

            def wait_rd(buf, sem):
                pltpu.make_async_copy(
                    tab_hbm.at[pl.ds(ebase, RCH), pl.ds(0, COLW)],
                    buf, sem).wait()

        def wait_sc(buf, sem):
            pltpu.make_async_copy(buf, acc.at[idxv.at[0]], sem).wait()

        for p in range(HID // COLW):
            _seg_zero(sid, bufa, acc, zstripe)
            plsc.subcore_barrier()
            rd(0, bufb, semb)

            def pair(k, c):
                j0 = 2 * k
                j1 = j0 + 1
                j2 = jnp.minimum(j1 + 1, nch - 1)
                wait_rd(bufb, semb)

                @pl.when(k > 0)
                def _():
                    wait_sc(bufa, sca)

                rd(j1, bufa, sema)
                pltpu.async_copy(bufb, acc.at[idxv.at[j0]], scb, add=True)
                wait_rd(bufa, sema)
                pltpu.async_copy(bufa, acc.at[idxv.at[j1]], sca, add=True)
                wait_sc(bufb, scb)
                rd(j2, bufb, semb)
                return c

            lax.fori_loop(0, nch // 2, pair, 0)
            wait_rd(bufb, semb)
            wait_sc(bufa, sca)
            plsc.subcore_barrier()
            w0 = sid * zstripe
            obase = cid * n_pad
            for off, sz in _chunks(zstripe, RCH):
                pltpu.sync_copy(acc.at[pl.ds(w0 + off, sz)],
                                bufa.at[pl.ds(0, sz)])
                pltpu.sync_copy(bufa.at[pl.ds(0, sz)],
                                out_hbm.at[pl.ds(obase + w0 + off, sz),
                                           pl.ds(p * COLW, COLW)])
            plsc.subcore_barrier()

    scratch = [
        pltpu.VMEM_SHARED((acc_rows, COLW), jnp.float32),
        pltpu.VMEM((RCH, COLW), jnp.float32),
        pltpu.VMEM((RCH, COLW), jnp.float32),
    ]
    if gather_mode:
        scratch.append(pltpu.VMEM((nch, RCH), jnp.int32))
    scratch += [
        pltpu.VMEM((nch, RCH), jnp.int32),
        pltpu.SemaphoreType.DMA,
        pltpu.SemaphoreType.DMA,
        pltpu.SemaphoreType.DMA,
        pltpu.SemaphoreType.DMA,
    ]
    k = pl.kernel(
        body,
        mesh=mesh,
        out_type=jax.ShapeDtypeStruct((2 * n_pad, HID), jnp.float32),
        scratch_types=scratch,
    )
    return k


def _segsum(msg, dst2d, n, n_pad, e_pad):
    r = _seg_kernel(False, n, n_pad, e_pad)(msg, dst2d)
    return r[:n_pad], r[n_pad:]


def _spmm(fs, src2d, dst2d, n, n_pad, e_pad):
    r = _seg_kernel(True, n, n_pad, e_pad)(fs, src2d, dst2d)
    return r[:n_pad], r[n_pad:]


def _dot(a, b):
    return jnp.dot(a, b, preferred_element_type=jnp.float32)


def _deg_isqrt(od16, id16):
    n_pad = od16.shape[0]
    blk = n_pad // 8

    def body(od_r, id_r, os_r, oi_r):
        os_r[:] = lax.rsqrt(
            jnp.maximum(jnp.sum(od_r[:], axis=1, keepdims=True), 1.0))
        oi_r[:] = lax.rsqrt(
            jnp.maximum(jnp.sum(id_r[:], axis=1, keepdims=True), 1.0))

    return pl.pallas_call(
        body,
        grid=(n_pad // blk,),
        in_specs=[
            pl.BlockSpec((blk, COLW), lambda i: (i, 0)),
            pl.BlockSpec((blk, COLW), lambda i: (i, 0)),
        ],
        out_specs=[
            pl.BlockSpec((blk, 1), lambda i: (i, 0)),
            pl.BlockSpec((blk, 1), lambda i: (i, 0)),
        ],
        out_shape=[
            jax.ShapeDtypeStruct((n_pad, 1), jnp.float32),
            jax.ShapeDtypeStruct((n_pad, 1), jnp.float32),
        ],
    )(od16, id16)


def _edge_mlp(hs, hd, rad, ef, wsrc, wdst, wr, wef, b1, w2, b2, ncoord=0):
    e_pad, dd = hs.shape
    blk = 1024
    has_ef = ef is not None
    emit_rad = rad is None

    def body(*refs):
        if has_ef:
            (hs_r, hd_r, ef_r, wsrc_r, wdst_r, wr_r, wef_r, b1_r,
             w2_r, b2_r, out_r, rad_r) = refs
        else:
            (hs_r, hd_r, rad_r, wsrc_r, wdst_r, wr_r, b1_r, w2_r, b2_r,
             out_r) = refs
        if emit_rad:
            d = hs_r[:] - hd_r[:]
            cmask = (lax.broadcasted_iota(jnp.int32, (1, dd), 1)
                     < ncoord).astype(jnp.float32)
            radv = jnp.sum(d * d * cmask, axis=1, keepdims=True)
            rad_r[:] = radv
        else:
            radv = rad_r[:]
        f = _dot(hs_r[:], wsrc_r[:]) + _dot(hd_r[:], wdst_r[:])
        f = f + radv * wr_r[:] + b1_r[:]
        if has_ef:
            f = f + _dot(ef_r[:], wef_r[:])
        m = _silu(f)
        out_r[:] = _silu(_dot(m, w2_r[:]) + b2_r[:])

    in_specs = [
        pl.BlockSpec((blk, dd), lambda i: (i, 0)),
        pl.BlockSpec((blk, dd), lambda i: (i, 0)),
    ]
    args = [hs, hd]
    if not emit_rad:
        in_specs.append(pl.BlockSpec((blk, 1), lambda i: (i, 0)))
        args.append(rad)
    if has_ef:
        in_specs.append(pl.BlockSpec((blk, ef.shape[1]), lambda i: (i, 0)))
        args.append(ef)
    full = [wsrc, wdst, wr] + ([wef] if has_ef else []) + [b1, w2, b2]
    for w in full:
        in_specs.append(pl.BlockSpec(w.shape, lambda i: (0, 0)))
        args.append(w)
    out_specs = pl.BlockSpec((blk, HID), lambda i: (i, 0))
    out_shape = jax.ShapeDtypeStruct((e_pad, HID), jnp.float32)
    if emit_rad:
        out_specs = [out_specs, pl.BlockSpec((blk, 1), lambda i: (i, 0))]
        out_shape = [out_shape, jax.ShapeDtypeStruct((e_pad, 1), jnp.float32)]
    return pl.pallas_call(
        body,
        grid=(e_pad // blk,),
        in_specs=in_specs,
        out_specs=out_specs,
        out_shape=out_shape,
    )(*args)


def _node_mlp(h, hn0, hn1, w1h, w1n, b1, w2, b2, oisq=None):
    n_pad, dh = h.shape
    blk = n_pad // 8
    scale = oisq is not None

    def body(*refs):
        if scale:
            (h_r, hn0_r, hn1_r, w1h_r, w1n_r, b1_r, w2_r, b2_r, oq_r, out_r,
             outs_r) = refs
        else:
            h_r, hn0_r, hn1_r, w1h_r, w1n_r, b1_r, w2_r, b2_r, out_r = refs
        hn_r = hn0_r[:] + hn1_r[:]
        a = _silu(_dot(h_r[:], w1h_r[:]) + _dot(hn_r, w1n_r[:]) + b1_r[:])
        o = _dot(a, w2_r[:]) + b2_r[:]
        out_r[:] = o
        if scale:
            outs_r[:] = o * oq_r[:]

    in_specs = [
        pl.BlockSpec((blk, dh), lambda i: (i, 0)),
        pl.BlockSpec((blk, HID), lambda i: (i, 0)),
        pl.BlockSpec((blk, HID), lambda i: (i, 0)),
        pl.BlockSpec(w1h.shape, lambda i: (0, 0)),
        pl.BlockSpec(w1n.shape, lambda i: (0, 0)),
        pl.BlockSpec(b1.shape, lambda i: (0, 0)),
        pl.BlockSpec(w2.shape, lambda i: (0, 0)),
        pl.BlockSpec(b2.shape, lambda i: (0, 0)),
    ]
    args = [h, hn0, hn1, w1h, w1n, b1, w2, b2]
    out_specs = pl.BlockSpec((blk, HID), lambda i: (i, 0))
    out_shape = jax.ShapeDtypeStruct((n_pad, HID), jnp.float32)
    if scale:
        in_specs.append(pl.BlockSpec((blk, 1), lambda i: (i, 0)))
        args.append(oisq)
        out_specs = [out_specs, pl.BlockSpec((blk, HID), lambda i: (i, 0))]
        out_shape = [out_shape,
                     jax.ShapeDtypeStruct((n_pad, HID), jnp.float32)]
    return pl.pallas_call(
        body,
        grid=(n_pad // blk,),
        in_specs=in_specs,
        out_specs=out_specs,
        out_shape=out_shape,
    )(*args)


def _gcn_update(agg0, agg1, f0, iisq, oisq, w, b, beta):
    n_pad = agg0.shape[0]
    blk = n_pad // 8

    def body(ag0_r, ag1_r, f0_r, iq_r, oq_r, w_r, b_r, out_r, outs_r):
        g = ((ag0_r[:] + ag1_r[:]) * iq_r[:]) * (1.0 - ALPHA) + f0_r[:] * ALPHA
        rst = g * (1.0 - beta) + _dot(g, w_r[:]) * beta + b_r[:]
        o = jnp.where(rst > 0, rst, jnp.exp(rst) - 1.0)
        out_r[:] = o
        outs_r[:] = o * oq_r[:]

    return pl.pallas_call(
        body,
        grid=(n_pad // blk,),
        in_specs=[
            pl.BlockSpec((blk, HID), lambda i: (i, 0)),
            pl.BlockSpec((blk, HID), lambda i: (i, 0)),
            pl.BlockSpec((blk, HID), lambda i: (i, 0)),
            pl.BlockSpec((blk, 1), lambda i: (i, 0)),
            pl.BlockSpec((blk, 1), lambda i: (i, 0)),
            pl.BlockSpec(w.shape, lambda i: (0, 0)),
            pl.BlockSpec(b.shape, lambda i: (0, 0)),
        ],
        out_specs=[
            pl.BlockSpec((blk, HID), lambda i: (i, 0)),
            pl.BlockSpec((blk, HID), lambda i: (i, 0)),
        ],
        out_shape=[
            jax.ShapeDtypeStruct((n_pad, HID), jnp.float32),
            jax.ShapeDtypeStruct((n_pad, HID), jnp.float32),
        ],
    )(agg0, agg1, f0, iisq, oisq, w, b)


def _ceil_to(x, m):
    return (x + m - 1) // m * m


def kernel(node_feats, coord, edge_feats, edge_index, params):
    n, din = node_feats.shape
    e = edge_index.shape[1]
    n_pad = _ceil_to(n, 2 * NS * RCH // 2)
    e_pad = _ceil_to(e, NW * CH)

    src = edge_index[0]
    dst = edge_index[1]
    pad_e = e_pad - e
    srcp = jnp.pad(src, (0, pad_e), constant_values=-1)
    dstp = jnp.pad(dst, (0, pad_e), constant_values=-1)
    srcg = jnp.maximum(srcp, 0)
    dstg = jnp.maximum(dstp, 0)
    src2d = srcg.reshape(e_pad // RCH, RCH)
    src2d_deg = srcp.reshape(e_pad // RCH, RCH)
    dst2d = dstp.reshape(e_pad // RCH, RCH)
    ones128 = jnp.zeros((CH, COLW), jnp.float32).at[:, 0].set(1.0)
    odp = _deg_partial(src2d_deg, ones128, n, n_pad, e_pad)
    idp = _deg_partial(dst2d, ones128, n, n_pad, e_pad)
    oisq, iisq = _deg_isqrt(odp[:n_pad] + odp[n_pad:], idp[:n_pad] + idp[n_pad:])
    efp = jnp.pad(edge_feats, ((0, pad_e), (0, 0)))

    ncoord = coord.shape[1]
    din_p = _ceil_to(din + ncoord, 128)
    h = jnp.pad(jnp.concatenate([coord, node_feats], axis=1),
                ((0, n_pad - n), (0, din_p - din - ncoord)))

    p0 = params["egnn0"]
    zc = jnp.zeros((ncoord, HID), jnp.float32)
    w1 = p0["edge1"]["W"]
    wsrc = jnp.pad(jnp.concatenate([zc, w1[:din]], axis=0),
                   ((0, din_p - din - ncoord), (0, 0)))
    wdst = jnp.pad(jnp.concatenate([zc, w1[din:2 * din]], axis=0),
                   ((0, din_p - din - ncoord), (0, 0)))
    wr = w1[2 * din:2 * din + 1]
    wef = w1[2 * din + 1:]
    hs, hd = _gather2(h, srcg, dstg, e_pad)
    msg, rad = _edge_mlp(hs, hd, None, efp, wsrc, wdst, wr, wef,
                         p0["edge1"]["b"][None], p0["edge2"]["W"],
                         p0["edge2"]["b"][None], ncoord=ncoord)
    hn0, hn1 = _segsum(msg, dst2d, n, n_pad, e_pad)
    n1 = p0["node1"]["W"]
    w1h = jnp.pad(jnp.concatenate([zc, n1[:din]], axis=0),
                  ((0, din_p - din - ncoord), (0, 0)))
    h = _node_mlp(h, hn0, hn1, w1h, n1[din:], p0["node1"]["b"][None],
                  p0["node2"]["W"], p0["node2"]["b"][None])
    h0 = h

    n_extra = len(params["egnn_layers"])
    feat_s = None
    for li, p in enumerate(params["egnn_layers"]):
        w1 = p["edge1"]["W"]
        hs, hd = _gather2(h, srcg, dstg, e_pad)
        msg = _edge_mlp(hs, hd, rad, None, w1[:HID], w1[HID:2 * HID],
                        w1[2 * HID:], None, p["edge1"]["b"][None],
                        p["edge2"]["W"], p["edge2"]["b"][None])
        hn0, hn1 = _segsum(msg, dst2d, n, n_pad, e_pad)
        n1 = p["node1"]["W"]
        last = li == n_extra - 1
        res = _node_mlp(h, hn0, hn1, n1[:HID], n1[HID:],
                        p["node1"]["b"][None], p["node2"]["W"],
                        p["node2"]["b"][None], oisq=oisq if last else None)
        if last:
            h, feat_s = res
        else:
            h = res

    feat = h
    for i, p in enumerate(params["gcn_layers"]):
        beta = float(np.log(1.0 / (i + 1) + 1.0))
        agg0, agg1 = _spmm(feat_s, src2d, dst2d, n, n_pad, e_pad)
        feat, feat_s = _gcn_update(agg0, agg1, h0, iisq, oisq, p["W"], p["b"][None],
                                   beta)
    return feat[:n]
